# baseline (speedup 1.0000x reference)
import jax
import jax.numpy as jnp
from jax.experimental import pallas as pl
from jax.experimental.pallas import tpu as pltpu


def _ax6(vals, pos):
    shape = [1] * 6
    shape[pos] = len(vals)
    return jnp.asarray(list(vals), jnp.int32).reshape(shape)


def _conv_block(w, hi0, hi1, ho0, ho1, W, in_cmajor=False):
    cout, cin = w.shape[0], w.shape[1]
    bf16 = jnp.bfloat16
    if in_cmajor:
        ci_p, hi_p, wi_p = 0, 1, 2
    else:
        hi_p, ci_p, wi_p = 0, 1, 2
    hi = _ax6(range(hi0, hi1), hi_p)
    ci = _ax6(range(cin), ci_p)
    wi = _ax6(range(W), wi_p)
    ho = _ax6(range(ho0, ho1), 3)
    wo = _ax6(range(W), 5)
    dims = [0] * 6
    dims[hi_p], dims[ci_p], dims[wi_p] = hi1 - hi0, cin, W
    dims[3], dims[4], dims[5] = ho1 - ho0, cout, W
    wb = w.astype(bf16)
    K = jnp.zeros(tuple(dims), bf16)
    arm_shape = [1] * 6
    arm_shape[ci_p], arm_shape[4] = cin, cout
    for dh in (-1, 0, 1):
        for dw in (-1, 0, 1):
            cond = (hi == ho + dh) & (wi == wo + dw)
            arm = wb[:, :, dh + 1, dw + 1].T.reshape(arm_shape)
            K = jnp.where(cond, arm, K)
    return K.reshape((hi1 - hi0) * cin * W, (ho1 - ho0) * cout * W)


def _pool_block(C, W, nh):
    W2 = W // 2
    hi = _ax6(range(nh), 0)
    ci = _ax6(range(C), 1)
    wi = _ax6(range(W), 2)
    ho = _ax6(range(nh // 2), 3)
    co = _ax6(range(C), 4)
    wo = _ax6(range(W2), 5)
    cond = (hi == 2 * ho) & (ci == co) & (wi == 2 * wo)
    S = jnp.where(cond, jnp.bfloat16(1), jnp.bfloat16(0))
    S = jnp.broadcast_to(S, (nh, C, W, nh // 2, C, W2))
    return S.reshape(nh * C * W, (nh // 2) * C * W2)


def _features_kernel(x_ref, buf_ref, o_ref):
    f32 = jnp.float32
    bf16 = jnp.bfloat16

    k0 = buf_ref[0:768, :]
    s1 = buf_ref[768:1792, 0:256]
    s2 = buf_ref[768:1792, 256:512]
    k1 = buf_ref[768:1280, 512:768]
    k3 = buf_ref[1280:1792, 512:768]
    k2 = buf_ref[768:1024, 768:1024]
    k4 = buf_ref[768:1024, 1024:1536]
    b0 = buf_ref[1792:1793, 0:2048].astype(f32)
    b1 = buf_ref[1800:1801, 0:256].astype(f32)
    b2 = buf_ref[1808:1809, 0:256].astype(f32)
    b3 = buf_ref[1816:1817, 0:256].astype(f32)
    b4 = buf_ref[1824:1825, 0:512].astype(f32)

    def dot(a, k):
        return jnp.dot(a, k, preferred_element_type=f32)

    def relu_pack(y, b):
        return jnp.maximum(y + b, 0.0).astype(bf16)

    def conv_blocked(src, k, b, nh, lanes_per_h):
        rows = lanes_per_h
        outs = []
        for t in range(nh // 2):
            i0, i1 = max(0, 2 * t - 1), min(nh, 2 * t + 3)
            lhs = src[:, i0 * rows:i1 * rows]
            r0 = rows if t == 0 else 0
            r1 = 3 * rows if t == nh // 2 - 1 else 4 * rows
            outs.append(relu_pack(dot(lhs, k[r0:r1, :]), b))
        return jnp.concatenate(outs, axis=1)

    def pool_maxes(y):
        a = jnp.maximum(y, jnp.concatenate([y[:, 1:], y[:, :1]], axis=1))
        return jnp.maximum(a, jnp.concatenate([a[:, 128:], a[:, :128]], axis=1))

    x = x_ref[...].astype(bf16)
    h = relu_pack(dot(x, k0), b0)

    h = conv_blocked(h, k1, b1, 16, 128)

    a = pool_maxes(h)
    p1 = jnp.concatenate(
        [dot(a[:, 0:1024], s1).astype(bf16),
         dot(a[:, 1024:2048], s1).astype(bf16)], axis=1)

    h = conv_blocked(p1, k2, b2, 8, 64)

    h = conv_blocked(h, k3, b3, 8, 128)

    p2 = dot(pool_maxes(h), s2).astype(bf16)

    o_ref[...] = jnp.maximum(dot(p2, k4) + b4, 0.0)


def kernel(x, w0, b0, w1, b1, w2, b2, w3, b3, w4, b4):
    N = x.shape[0]
    f32, bf16 = jnp.float32, jnp.bfloat16

    xf = x.reshape(N, 768)

    K0 = _conv_block(w0, 0, 16, 0, 16, 16, in_cmajor=True)
    K1 = _conv_block(w1, -1, 3, 0, 2, 16)
    K2 = _conv_block(w2, -1, 3, 0, 2, 8)
    K3 = _conv_block(w3, -1, 3, 0, 2, 8)
    K4h = _conv_block(w4, 0, 4, 0, 4, 4)
    K4 = K4h.reshape(256, 4, 32, 4).transpose(0, 2, 1, 3).reshape(256, 512)
    S1 = _pool_block(8, 16, 8)
    S2 = _pool_block(16, 8, 8)

    def zs(r, c):
        return jnp.zeros((r, c), bf16)

    def brow(b, rep, tile, width):
        row = jnp.tile(jnp.repeat(b, rep), tile).reshape(1, -1).astype(bf16)
        return jnp.concatenate(
            [jnp.concatenate([row, zs(1, 2048 - width)], axis=1), zs(7, 2048)],
            axis=0)

    mid = jnp.concatenate(
        [S1, S2,
         jnp.concatenate([K1, K3], axis=0),
         jnp.concatenate([K2, zs(768, 256)], axis=0),
         jnp.concatenate([K4, zs(768, 512)], axis=0),
         zs(1024, 512)], axis=1)
    buf = jnp.concatenate(
        [K0, mid,
         brow(b0, 16, 16, 2048), brow(b1, 16, 2, 256),
         brow(b2, 8, 2, 256), brow(b3, 8, 2, 256),
         brow(b4, 16, 1, 512)], axis=0)

    NB = 512 if N % 512 == 0 else N
    grid = (N // NB,)

    out = pl.pallas_call(
        _features_kernel,
        out_shape=jax.ShapeDtypeStruct((N, 512), f32),
        grid=grid,
        in_specs=[pl.BlockSpec((NB, 768), lambda i: (i, 0)),
                  pl.BlockSpec(buf.shape, lambda i: (0, 0))],
        out_specs=pl.BlockSpec((NB, 512), lambda i: (i, 0)),
        compiler_params=pltpu.CompilerParams(
            dimension_semantics=("arbitrary",),
            vmem_limit_bytes=64 * 1024 * 1024),
    )(xf, buf)
    return out.reshape(N, 32, 4, 4)


# speedup vs baseline: 2.0089x; 2.0089x over previous
import jax
import jax.numpy as jnp
from jax.experimental import pallas as pl
from jax.experimental.pallas import tpu as pltpu


def _conv_block(w, hi0, hi1, nho, W, in_cmajor=False, out_cmajor=False):
    cout, cin = w.shape[0], w.shape[1]
    bf16 = jnp.bfloat16
    L = nho * cout * W
    l = jnp.arange(L)
    if out_cmajor:
        co_l, ho_l, wo_l = l // (nho * W), (l // W) % nho, l % W
    else:
        ho_l, co_l, wo_l = l // (cout * W), (l // W) % cout, l % W
    nhi = hi1 - hi0
    if in_cmajor:
        d0, d1 = cin, nhi
        hi = jnp.arange(hi0, hi1).reshape(1, nhi, 1, 1)
    else:
        d0, d1 = nhi, cin
        hi = jnp.arange(hi0, hi1).reshape(nhi, 1, 1, 1)
    wi = jnp.arange(W).reshape(1, 1, W, 1)
    wb = w.astype(bf16)
    K = jnp.zeros((d0, d1, W, L), bf16)
    for dh in (-1, 0, 1):
        for dw in (-1, 0, 1):
            cond = (hi == ho_l + dh) & (wi == wo_l + dw)
            wt = wb[:, :, dh + 1, dw + 1].T
            if out_cmajor:
                arm = jnp.repeat(wt, nho * W, axis=1)
            else:
                arm = jnp.tile(jnp.repeat(wt, W, axis=1), (1, nho))
            arm = arm.reshape((cin, 1, 1, L) if in_cmajor else (1, cin, 1, L))
            K = jnp.where(cond, arm, K)
    return K.reshape(nhi * cin * W, L)


def _pool_block(C, W, nh):
    W2 = W // 2
    R, L = nh * C * W, (nh // 2) * C * W2
    r = jnp.arange(R)[:, None]
    hi, ci, wi = r // (C * W), (r // W) % C, r % W
    lc = jnp.arange(L)[None, :]
    ho, co, wo = lc // (C * W2), (lc // W2) % C, lc % W2
    cond = (hi == 2 * ho) & (ci == co) & (wi == 2 * wo)
    return cond.astype(jnp.bfloat16)


def _bias_buf(b0, b1, b2, b3, b4):
    r = jnp.arange(8)[:, None]
    l = jnp.arange(2048)[None, :]
    e0 = jnp.tile(jnp.repeat(b0, 16), 16)[None, :]
    e1 = jnp.tile(jnp.repeat(b1, 16), 16)[None, :]
    e2 = jnp.tile(jnp.repeat(b2, 8), 16)[None, :]
    e3 = jnp.tile(jnp.repeat(b3, 8), 16)[None, :]
    e4 = jnp.tile(jnp.repeat(b4, 16), 4)[None, :]
    z = jnp.zeros((1, 2048), jnp.float32)
    buf = jnp.where(r == 0, e0,
          jnp.where(r == 1, jnp.where(l < 256, e1, z),
          jnp.where(r == 2, jnp.where(l < 256, e2, z),
          jnp.where(r == 3, jnp.where(l < 256, e3, z),
          jnp.where(r == 4, jnp.where(l < 512, e4, z), z)))))
    return buf.astype(jnp.bfloat16)


def _features_kernel(x_ref, k0, k1, k2, k3, k4, s1, s2, bb, o_ref):
    f32 = jnp.float32
    bf16 = jnp.bfloat16

    b0 = bb[0:1, 0:2048].astype(f32)
    b1 = bb[1:2, 0:256].astype(f32)
    b2 = bb[2:3, 0:256].astype(f32)
    b3 = bb[3:4, 0:256].astype(f32)
    b4 = bb[4:5, 0:512].astype(f32)

    def dot(a, k):
        return jnp.dot(a, k, preferred_element_type=f32)

    def relu_pack(y, b):
        return jnp.maximum(y + b, 0.0).astype(bf16)

    def conv_blocked(src, k, b, nh, lanes_per_h):
        rows = lanes_per_h
        outs = []
        for t in range(nh // 2):
            i0, i1 = max(0, 2 * t - 1), min(nh, 2 * t + 3)
            lhs = src[:, i0 * rows:i1 * rows]
            r0 = rows if t == 0 else 0
            r1 = 3 * rows if t == nh // 2 - 1 else 4 * rows
            outs.append(relu_pack(dot(lhs, k[r0:r1, :]), b))
        return jnp.concatenate(outs, axis=1)

    def pool_maxes(y):
        a = jnp.maximum(y, jnp.concatenate([y[:, 1:], y[:, :1]], axis=1))
        return jnp.maximum(a, jnp.concatenate([a[:, 128:], a[:, :128]], axis=1))

    x = x_ref[...].astype(bf16)
    h = relu_pack(dot(x, k0[...]), b0)

    h = conv_blocked(h, k1[...], b1, 16, 128)

    a = pool_maxes(h)
    p1 = jnp.concatenate(
        [dot(a[:, 0:1024], s1[...]).astype(bf16),
         dot(a[:, 1024:2048], s1[...]).astype(bf16)], axis=1)

    h = conv_blocked(p1, k2[...], b2, 8, 64)

    h = conv_blocked(h, k3[...], b3, 8, 128)

    p2 = dot(pool_maxes(h), s2[...]).astype(bf16)

    o_ref[...] = jnp.maximum(dot(p2, k4[...]) + b4, 0.0)


def kernel(x, w0, b0, w1, b1, w2, b2, w3, b3, w4, b4):
    N = x.shape[0]
    f32 = jnp.float32

    xf = x.reshape(N, 768)

    K0 = _conv_block(w0, 0, 16, 16, 16, in_cmajor=True)
    K1 = _conv_block(w1, -1, 3, 2, 16)
    K2 = _conv_block(w2, -1, 3, 2, 8)
    K3 = _conv_block(w3, -1, 3, 2, 8)
    K4 = _conv_block(w4, 0, 4, 4, 4, out_cmajor=True)
    S1 = _pool_block(8, 16, 8)
    S2 = _pool_block(16, 8, 8)
    BB = _bias_buf(b0, b1, b2, b3, b4)

    NB = 512 if N % 512 == 0 else N
    grid = (N // NB,)

    consts = [K0, K1, K2, K3, K4, S1, S2, BB]

    def cspec(a):
        return pl.BlockSpec(a.shape, lambda i: (0, 0))

    out = pl.pallas_call(
        _features_kernel,
        out_shape=jax.ShapeDtypeStruct((N, 512), f32),
        grid=grid,
        in_specs=[pl.BlockSpec((NB, 768), lambda i: (i, 0))] +
                 [cspec(a) for a in consts],
        out_specs=pl.BlockSpec((NB, 512), lambda i: (i, 0)),
        compiler_params=pltpu.CompilerParams(
            dimension_semantics=("arbitrary",),
            vmem_limit_bytes=64 * 1024 * 1024),
    )(xf, *consts)
    return out.reshape(N, 32, 4, 4)


# speedup vs baseline: 2.0578x; 1.0243x over previous
import jax
import jax.numpy as jnp
from jax.experimental import pallas as pl
from jax.experimental.pallas import tpu as pltpu


def _conv_block(w, hi0, hi1, nho, W, in_cmajor=False, out_cmajor=False):
    cout, cin = w.shape[0], w.shape[1]
    bf16 = jnp.bfloat16
    L = nho * cout * W
    l = jnp.arange(L)
    if out_cmajor:
        co_l, ho_l, wo_l = l // (nho * W), (l // W) % nho, l % W
    else:
        ho_l, co_l, wo_l = l // (cout * W), (l // W) % cout, l % W
    nhi = hi1 - hi0
    if in_cmajor:
        d0, d1 = cin, nhi
        hi = jnp.arange(hi0, hi1).reshape(1, nhi, 1, 1)
    else:
        d0, d1 = nhi, cin
        hi = jnp.arange(hi0, hi1).reshape(nhi, 1, 1, 1)
    wi = jnp.arange(W).reshape(1, 1, W, 1)
    wb = w.astype(bf16)
    K = jnp.zeros((d0, d1, W, L), bf16)
    for dh in (-1, 0, 1):
        for dw in (-1, 0, 1):
            cond = (hi == ho_l + dh) & (wi == wo_l + dw)
            wt = wb[:, :, dh + 1, dw + 1].T
            if out_cmajor:
                arm = jnp.repeat(wt, nho * W, axis=1)
            else:
                arm = jnp.tile(jnp.repeat(wt, W, axis=1), (1, nho))
            arm = arm.reshape((cin, 1, 1, L) if in_cmajor else (1, cin, 1, L))
            K = jnp.where(cond, arm, K)
    return K.reshape(nhi * cin * W, L)


def _pool_block(C, W, nh):
    W2 = W // 2
    R, L = nh * C * W, (nh // 2) * C * W2
    r = jnp.arange(R)[:, None]
    hi, ci, wi = r // (C * W), (r // W) % C, r % W
    lc = jnp.arange(L)[None, :]
    ho, co, wo = lc // (C * W2), (lc // W2) % C, lc % W2
    cond = (hi == 2 * ho) & (ci == co) & (wi == 2 * wo)
    return cond.astype(jnp.bfloat16)


def _bias_buf(b0, b1, b2, b3, b4):
    r = jnp.arange(8)[:, None]
    l = jnp.arange(2048)[None, :]
    e0 = jnp.tile(jnp.repeat(b0, 16), 16)[None, :]
    e1 = jnp.tile(jnp.repeat(b1, 16), 16)[None, :]
    e2 = jnp.tile(jnp.repeat(b2, 8), 16)[None, :]
    e3 = jnp.tile(jnp.repeat(b3, 8), 16)[None, :]
    e4 = jnp.tile(jnp.repeat(b4, 16), 4)[None, :]
    z = jnp.zeros((1, 2048), jnp.float32)
    buf = jnp.where(r == 0, e0,
          jnp.where(r == 1, jnp.where(l < 256, e1, z),
          jnp.where(r == 2, jnp.where(l < 256, e2, z),
          jnp.where(r == 3, jnp.where(l < 256, e3, z),
          jnp.where(r == 4, jnp.where(l < 512, e4, z), z)))))
    return buf.astype(jnp.bfloat16)


def _features_kernel(x_ref, k0, k1, k2, k3, k4, s1, s2, bb, o_ref):
    f32 = jnp.float32
    bf16 = jnp.bfloat16

    b0 = bb[0:1, 0:2048].astype(f32)
    b1 = bb[1:2, 0:256].astype(f32)
    b2 = bb[2:3, 0:256].astype(f32)
    b3 = bb[3:4, 0:256].astype(f32)
    b4 = bb[4:5, 0:512].astype(f32)

    def dot(a, k):
        return jnp.dot(a, k, preferred_element_type=f32)

    def relu_pack(y, b):
        return jnp.maximum(y + b, 0.0).astype(bf16)

    def conv_blocked(src, k, b, nh, lanes_per_h):
        rows = lanes_per_h
        outs = []
        for t in range(nh // 2):
            i0, i1 = max(0, 2 * t - 1), min(nh, 2 * t + 3)
            lhs = src[:, i0 * rows:i1 * rows]
            r0 = rows if t == 0 else 0
            r1 = 3 * rows if t == nh // 2 - 1 else 4 * rows
            outs.append(relu_pack(dot(lhs, k[r0:r1, :]), b))
        return jnp.concatenate(outs, axis=1)

    def pool_maxes(y):
        a = jnp.maximum(y, jnp.concatenate([y[:, 1:], y[:, :1]], axis=1))
        return jnp.maximum(a, jnp.concatenate([a[:, 128:], a[:, :128]], axis=1))

    x = x_ref[...].astype(bf16)
    h = relu_pack(dot(x, k0[...]), b0)

    h = conv_blocked(h, k1[...], b1, 16, 128)

    a = pool_maxes(h)
    p1 = jnp.concatenate(
        [dot(a[:, 0:1024], s1[...]).astype(bf16),
         dot(a[:, 1024:2048], s1[...]).astype(bf16)], axis=1)

    h = conv_blocked(p1, k2[...], b2, 8, 64)

    h = conv_blocked(h, k3[...], b3, 8, 128)

    p2 = dot(pool_maxes(h), s2[...]).astype(bf16)

    o_ref[...] = jnp.maximum(dot(p2, k4[...]) + b4, 0.0)


def kernel(x, w0, b0, w1, b1, w2, b2, w3, b3, w4, b4):
    N = x.shape[0]
    f32 = jnp.float32

    xf = x.reshape(N, 768)

    K0 = _conv_block(w0, 0, 16, 16, 16, in_cmajor=True)
    K1 = _conv_block(w1, -1, 3, 2, 16)
    K2 = _conv_block(w2, -1, 3, 2, 8)
    K3 = _conv_block(w3, -1, 3, 2, 8)
    K4 = _conv_block(w4, 0, 4, 4, 4, out_cmajor=True)
    S1 = _pool_block(8, 16, 8)
    S2 = _pool_block(16, 8, 8)
    BB = _bias_buf(b0, b1, b2, b3, b4)

    NB = 1024 if N % 1024 == 0 else N
    grid = (N // NB,)

    consts = [K0, K1, K2, K3, K4, S1, S2, BB]

    def cspec(a):
        return pl.BlockSpec(a.shape, lambda i: (0, 0))

    out = pl.pallas_call(
        _features_kernel,
        out_shape=jax.ShapeDtypeStruct((N, 512), f32),
        grid=grid,
        in_specs=[pl.BlockSpec((NB, 768), lambda i: (i, 0))] +
                 [cspec(a) for a in consts],
        out_specs=pl.BlockSpec((NB, 512), lambda i: (i, 0)),
        compiler_params=pltpu.CompilerParams(
            dimension_semantics=("arbitrary",),
            vmem_limit_bytes=64 * 1024 * 1024),
    )(xf, *consts)
    return out.reshape(N, 32, 4, 4)
